# baseline (speedup 1.0000x reference)
import functools

import jax
import jax.numpy as jnp
from jax import lax
from jax.experimental import pallas as pl
from jax.experimental.pallas import tpu as pltpu
from jax.experimental.pallas import tpu_sc as plsc

_NUM_EMB = 1024
_DIM = 64
_PDIM = 32
_N = 32768
_R = 1024
_NB = _N // _R
_NB_HI = 8
_NB_LO = _NB - _NB_HI

_NW = 32
_BPW = _N // _NW
_CHUNK = 128
_NCH = _BPW // _CHUNK


def _embproj_kernel(emb_ref, w_ref, b_ref, embp_ref, embn_ref):
    emb_p = jax.lax.dot_general(
        emb_ref[:], w_ref[:], (((1,), (1,)), ((), ())),
        preferred_element_type=jnp.float32) + b_ref[:]
    embp_ref[:] = emb_p
    embn_ref[:] = jnp.sum(emb_p * emb_p, axis=1)[None, :]


def _compute_closest(z_ref, embp_ref, embn_ref, w_ref, b_ref):
    z_p = jax.lax.dot_general(
        z_ref[:], w_ref[:], (((1,), (1,)), ((), ())),
        preferred_element_type=jnp.float32) + b_ref[:]
    rowsq = jnp.sum(z_p * z_p, axis=1, keepdims=True)
    cross = jax.lax.dot_general(
        z_p, embp_ref[:], (((1,), (1,)), ((), ())),
        preferred_element_type=jnp.float32)
    dist = (rowsq + embn_ref[:]) - 2.0 * cross
    minv = jnp.min(dist, axis=1, keepdims=True)
    iota = jax.lax.broadcasted_iota(jnp.int32, (_R, _NUM_EMB), 1)
    masked = jnp.where(dist == minv, iota, jnp.int32(2 ** 30))
    closest = jnp.min(masked, axis=1)
    return closest, iota


def _cl_kernel(z_ref, embp_ref, embn_ref, w_ref, b_ref, cl_ref):
    closest, _ = _compute_closest(z_ref, embp_ref, embn_ref, w_ref, b_ref)
    cl_ref[:, :] = closest.reshape(_R // _CHUNK, _CHUNK)


def _oh_cl_kernel(z_ref, embp_ref, embn_ref, w_ref, b_ref, clin_ref,
                  oh_ref, cl_ref):
    del clin_ref
    closest, iota = _compute_closest(z_ref, embp_ref, embn_ref, w_ref, b_ref)
    oh_ref[:] = (iota == closest[:, None]).astype(jnp.float32)
    cl_ref[:, :] = closest.reshape(_R // _CHUNK, _CHUNK)


def _oh_kernel(z_ref, embp_ref, embn_ref, w_ref, b_ref, ohin_ref, oh_ref):
    del ohin_ref
    closest, iota = _compute_closest(z_ref, embp_ref, embn_ref, w_ref, b_ref)
    oh_ref[:] = (iota == closest[:, None]).astype(jnp.float32)


_mesh = plsc.VectorSubcoreMesh(core_axis_name="c", subcore_axis_name="s",
                               num_cores=2, num_subcores=16)


@functools.partial(
    pl.kernel,
    out_type=jax.ShapeDtypeStruct((_N, _DIM), jnp.float32),
    mesh=_mesh,
    scratch_types=[
        pltpu.VMEM((_NCH, _CHUNK), jnp.int32),
    ] + [pltpu.VMEM((_CHUNK, _DIM), jnp.float32) for _ in range(_NCH)] + [
        pltpu.SemaphoreType.DMA,
        pltpu.SemaphoreType.DMA,
    ],
    compiler_params=pltpu.CompilerParams(use_tc_tiling_on_sc=False),
)
def _gather_kernel(cl_hbm, table_hbm, out_hbm, idx_v, *rest):
    bufs, (sem_g, sem_w) = rest[:_NCH], rest[_NCH:]
    wid = lax.axis_index("s") * 2 + lax.axis_index("c")
    base = wid * _BPW
    pltpu.sync_copy(cl_hbm.at[pl.ds(wid * _NCH, _NCH)], idx_v)
    gcps = [pltpu.async_copy(table_hbm.at[idx_v.at[c]], bufs[c], sem_g)
            for c in range(_NCH)]
    wcps = []
    for c in range(_NCH):
        gcps[c].wait()
        wcps.append(pltpu.async_copy(
            bufs[c], out_hbm.at[pl.ds(base + c * _CHUNK, _CHUNK)], sem_w))
    for w in wcps:
        w.wait()


_CONST_SPECS = [
    pl.BlockSpec((_NUM_EMB, _PDIM), lambda i: (0, 0)),
    pl.BlockSpec((1, _NUM_EMB), lambda i: (0, 0)),
    pl.BlockSpec((_PDIM, _DIM), lambda i: (0, 0)),
    pl.BlockSpec((1, _PDIM), lambda i: (0, 0)),
]


def kernel(z, embeddings, W, b):
    b2 = b.reshape(1, _PDIM)
    emb_p, embn = pl.pallas_call(
        _embproj_kernel,
        out_shape=[
            jax.ShapeDtypeStruct((_NUM_EMB, _PDIM), jnp.float32),
            jax.ShapeDtypeStruct((1, _NUM_EMB), jnp.float32),
        ],
    )(embeddings, W, b2)

    cl_a = pl.pallas_call(
        _cl_kernel,
        grid=(_NB_HI,),
        in_specs=[pl.BlockSpec((_R, _DIM), lambda j: (_NB_LO + j, 0))]
        + _CONST_SPECS,
        out_specs=pl.BlockSpec((_R // _CHUNK, _CHUNK),
                               lambda j: (_NB_LO + j, 0)),
        out_shape=jax.ShapeDtypeStruct((_N // _CHUNK, _CHUNK), jnp.int32),
    )(z, emb_p, embn, W, b2)

    oh_b, cl = pl.pallas_call(
        _oh_cl_kernel,
        grid=(_NB_LO,),
        in_specs=[pl.BlockSpec((_R, _DIM), lambda i: (i, 0))]
        + _CONST_SPECS
        + [pl.BlockSpec((8, _CHUNK), lambda i: (0, 0))],
        out_specs=[
            pl.BlockSpec((_R, _NUM_EMB), lambda i: (i, 0)),
            pl.BlockSpec((_R // _CHUNK, _CHUNK), lambda i: (i, 0)),
        ],
        out_shape=[
            jax.ShapeDtypeStruct((_N, _NUM_EMB), jnp.float32),
            jax.ShapeDtypeStruct((_N // _CHUNK, _CHUNK), jnp.int32),
        ],
        input_output_aliases={5: 1},
    )(z, emb_p, embn, W, b2, cl_a)

    quantized = _gather_kernel(cl, embeddings)

    one_hot = pl.pallas_call(
        _oh_kernel,
        grid=(_NB_HI,),
        in_specs=[pl.BlockSpec((_R, _DIM), lambda j: (_NB_LO + j, 0))]
        + _CONST_SPECS
        + [pl.BlockSpec((8, _CHUNK), lambda j: (0, 0))],
        out_specs=pl.BlockSpec((_R, _NUM_EMB), lambda j: (_NB_LO + j, 0)),
        out_shape=jax.ShapeDtypeStruct((_N, _NUM_EMB), jnp.float32),
        input_output_aliases={5: 0},
    )(z, emb_p, embn, W, b2, oh_b)

    return (quantized, one_hot)


# speedup vs baseline: 1.0043x; 1.0043x over previous
import functools

import jax
import jax.numpy as jnp
from jax import lax
from jax.experimental import pallas as pl
from jax.experimental.pallas import tpu as pltpu
from jax.experimental.pallas import tpu_sc as plsc

_NUM_EMB = 1024
_DIM = 64
_PDIM = 32
_N = 32768
_R = 1024
_NB = _N // _R
_NB_HI = 8
_NB_LO = _NB - _NB_HI
_N_LO = _NB_LO * _R
_N_HI = _NB_HI * _R

_NW = 32
_CHUNK = 128
_NCH_HI = _N_HI // _NW // _CHUNK
_NCH_LO = _N_LO // _NW // _CHUNK


def _embproj_kernel(emb_ref, w_ref, b_ref, embp_ref, embn_ref):
    emb_p = jax.lax.dot_general(
        emb_ref[:], w_ref[:], (((1,), (1,)), ((), ())),
        preferred_element_type=jnp.float32) + b_ref[:]
    embp_ref[:] = emb_p
    embn_ref[:] = jnp.sum(emb_p * emb_p, axis=1)[None, :]


def _compute_closest(z_ref, embp_ref, embn_ref, w_ref, b_ref):
    z_p = jax.lax.dot_general(
        z_ref[:], w_ref[:], (((1,), (1,)), ((), ())),
        preferred_element_type=jnp.float32) + b_ref[:]
    rowsq = jnp.sum(z_p * z_p, axis=1, keepdims=True)
    cross = jax.lax.dot_general(
        z_p, embp_ref[:], (((1,), (1,)), ((), ())),
        preferred_element_type=jnp.float32)
    dist = (rowsq + embn_ref[:]) - 2.0 * cross
    minv = jnp.min(dist, axis=1, keepdims=True)
    iota = jax.lax.broadcasted_iota(jnp.int32, (_R, _NUM_EMB), 1)
    masked = jnp.where(dist == minv, iota, jnp.int32(2 ** 30))
    closest = jnp.min(masked, axis=1)
    return closest, iota


def _cl_kernel(z_ref, embp_ref, embn_ref, w_ref, b_ref, cl_ref):
    closest, _ = _compute_closest(z_ref, embp_ref, embn_ref, w_ref, b_ref)
    cl_ref[:, :] = closest.reshape(_R // _CHUNK, _CHUNK)


def _oh_cl_kernel(z_ref, embp_ref, embn_ref, w_ref, b_ref, oh_ref, cl_ref):
    closest, iota = _compute_closest(z_ref, embp_ref, embn_ref, w_ref, b_ref)
    oh_ref[:] = (iota == closest[:, None]).astype(jnp.float32)
    cl_ref[:, :] = closest.reshape(_R // _CHUNK, _CHUNK)


def _oh_kernel(z_ref, embp_ref, embn_ref, w_ref, b_ref, ohin_ref, oh_ref):
    del ohin_ref
    closest, iota = _compute_closest(z_ref, embp_ref, embn_ref, w_ref, b_ref)
    oh_ref[:] = (iota == closest[:, None]).astype(jnp.float32)


_mesh = plsc.VectorSubcoreMesh(core_axis_name="c", subcore_axis_name="s",
                               num_cores=2, num_subcores=16)
_SC_PARAMS = pltpu.CompilerParams(use_tc_tiling_on_sc=False)


@functools.partial(
    pl.kernel,
    out_type=jax.ShapeDtypeStruct((_N_HI, _DIM), jnp.float32),
    mesh=_mesh,
    scratch_types=[
        pltpu.VMEM((_NCH_HI, _CHUNK), jnp.int32),
    ] + [pltpu.VMEM((_CHUNK, _DIM), jnp.float32) for _ in range(_NCH_HI)] + [
        pltpu.SemaphoreType.DMA,
        pltpu.SemaphoreType.DMA,
    ],
    compiler_params=_SC_PARAMS,
)
def _gather_hi_kernel(cl_hbm, table_hbm, out_hbm, idx_v, *rest):
    bufs, (sem_g, sem_w) = rest[:_NCH_HI], rest[_NCH_HI:]
    wid = lax.axis_index("s") * 2 + lax.axis_index("c")
    pltpu.sync_copy(cl_hbm.at[pl.ds(wid * _NCH_HI, _NCH_HI)], idx_v)
    gcps = [pltpu.async_copy(table_hbm.at[idx_v.at[c]], bufs[c], sem_g)
            for c in range(_NCH_HI)]
    wcps = []
    for c in range(_NCH_HI):
        gcps[c].wait()
        wcps.append(pltpu.async_copy(
            bufs[c],
            out_hbm.at[pl.ds((wid * _NCH_HI + c) * _CHUNK, _CHUNK)], sem_w))
    for w in wcps:
        w.wait()


@functools.partial(
    pl.kernel,
    out_type=jax.ShapeDtypeStruct((_N, _DIM), jnp.float32),
    mesh=_mesh,
    scratch_types=[
        pltpu.VMEM((_NCH_LO, _CHUNK), jnp.int32),
    ] + [pltpu.VMEM((_CHUNK, _DIM), jnp.float32)
         for _ in range(_NCH_LO + 2)] + [
        pltpu.SemaphoreType.DMA,
        pltpu.SemaphoreType.DMA,
    ],
    compiler_params=_SC_PARAMS,
)
def _gather_lo_kernel(cl_hbm, q_hi_hbm, table_hbm, out_hbm, idx_v, *rest):
    bufs, cbufs, (sem_g, sem_w) = (
        rest[:_NCH_LO], rest[_NCH_LO:_NCH_LO + 2], rest[_NCH_LO + 2:])
    wid = lax.axis_index("s") * 2 + lax.axis_index("c")
    pltpu.sync_copy(cl_hbm.at[pl.ds(wid * _NCH_LO, _NCH_LO)], idx_v)
    gcps = [pltpu.async_copy(table_hbm.at[idx_v.at[c]], bufs[c], sem_g)
            for c in range(_NCH_LO)]
    wcps = []
    for k in range(_NCH_HI):
        src = (wid * _NCH_HI + k) * _CHUNK
        pltpu.sync_copy(q_hi_hbm.at[pl.ds(src, _CHUNK)], cbufs[k])
        wcps.append(pltpu.async_copy(
            cbufs[k], out_hbm.at[pl.ds(_N_LO + src, _CHUNK)], sem_w))
    for c in range(_NCH_LO):
        gcps[c].wait()
        wcps.append(pltpu.async_copy(
            bufs[c],
            out_hbm.at[pl.ds((wid * _NCH_LO + c) * _CHUNK, _CHUNK)], sem_w))
    for w in wcps:
        w.wait()


_CONST_SPECS = [
    pl.BlockSpec((_NUM_EMB, _PDIM), lambda i: (0, 0)),
    pl.BlockSpec((1, _NUM_EMB), lambda i: (0, 0)),
    pl.BlockSpec((_PDIM, _DIM), lambda i: (0, 0)),
    pl.BlockSpec((1, _PDIM), lambda i: (0, 0)),
]


def kernel(z, embeddings, W, b):
    b2 = b.reshape(1, _PDIM)
    emb_p, embn = pl.pallas_call(
        _embproj_kernel,
        out_shape=[
            jax.ShapeDtypeStruct((_NUM_EMB, _PDIM), jnp.float32),
            jax.ShapeDtypeStruct((1, _NUM_EMB), jnp.float32),
        ],
    )(embeddings, W, b2)

    cl_hi = pl.pallas_call(
        _cl_kernel,
        grid=(_NB_HI,),
        in_specs=[pl.BlockSpec((_R, _DIM), lambda j: (_NB_LO + j, 0))]
        + _CONST_SPECS,
        out_specs=pl.BlockSpec((_R // _CHUNK, _CHUNK), lambda j: (j, 0)),
        out_shape=jax.ShapeDtypeStruct((_N_HI // _CHUNK, _CHUNK), jnp.int32),
    )(z, emb_p, embn, W, b2)

    q_hi = _gather_hi_kernel(cl_hi, embeddings)

    oh_b, cl_lo = pl.pallas_call(
        _oh_cl_kernel,
        grid=(_NB_LO,),
        in_specs=[pl.BlockSpec((_R, _DIM), lambda i: (i, 0))]
        + _CONST_SPECS,
        out_specs=[
            pl.BlockSpec((_R, _NUM_EMB), lambda i: (i, 0)),
            pl.BlockSpec((_R // _CHUNK, _CHUNK), lambda i: (i, 0)),
        ],
        out_shape=[
            jax.ShapeDtypeStruct((_N, _NUM_EMB), jnp.float32),
            jax.ShapeDtypeStruct((_N_LO // _CHUNK, _CHUNK), jnp.int32),
        ],
    )(z, emb_p, embn, W, b2)

    quantized = _gather_lo_kernel(cl_lo, q_hi, embeddings)

    one_hot = pl.pallas_call(
        _oh_kernel,
        grid=(_NB_HI,),
        in_specs=[pl.BlockSpec((_R, _DIM), lambda j: (_NB_LO + j, 0))]
        + _CONST_SPECS
        + [pl.BlockSpec((8, _CHUNK), lambda j: (0, 0))],
        out_specs=pl.BlockSpec((_R, _NUM_EMB), lambda j: (_NB_LO + j, 0)),
        out_shape=jax.ShapeDtypeStruct((_N, _NUM_EMB), jnp.float32),
        input_output_aliases={5: 0},
    )(z, emb_p, embn, W, b2, oh_b)

    return (quantized, one_hot)


# speedup vs baseline: 1.0635x; 1.0590x over previous
import functools

import jax
import jax.numpy as jnp
from jax import lax
from jax.experimental import pallas as pl
from jax.experimental.pallas import tpu as pltpu
from jax.experimental.pallas import tpu_sc as plsc

_NUM_EMB = 1024
_DIM = 64
_PDIM = 32
_N = 32768
_R = 1024

_NW = 32
_BPW = _N // _NW
_CHUNK = 128
_NCH = _BPW // _CHUNK


def _embproj_kernel(emb_ref, w_ref, b_ref, embp_ref, embn_ref):
    emb_p = jax.lax.dot_general(
        emb_ref[:], w_ref[:], (((1,), (1,)), ((), ())),
        preferred_element_type=jnp.float32) + b_ref[:]
    embp_ref[:] = emb_p
    embn_ref[:] = jnp.sum(emb_p * emb_p, axis=1)[None, :]


def _closest_kernel(z_ref, embp_ref, embn_ref, w_ref, b_ref, oh_ref, cl_ref):
    z_p = jax.lax.dot_general(
        z_ref[:], w_ref[:], (((1,), (1,)), ((), ())),
        preferred_element_type=jnp.float32) + b_ref[:]
    rowsq = jnp.sum(z_p * z_p, axis=1, keepdims=True)
    cross = jax.lax.dot_general(
        z_p, embp_ref[:], (((1,), (1,)), ((), ())),
        preferred_element_type=jnp.float32)
    dist = (rowsq + embn_ref[:]) - 2.0 * cross
    minv = jnp.min(dist, axis=1, keepdims=True)
    iota = jax.lax.broadcasted_iota(jnp.int32, (_R, _NUM_EMB), 1)
    masked = jnp.where(dist == minv, iota, jnp.int32(2 ** 30))
    closest = jnp.min(masked, axis=1)
    oh_ref[:] = (iota == closest[:, None]).astype(jnp.float32)
    cl_ref[:, :] = closest.reshape(_R // _CHUNK, _CHUNK)


_mesh = plsc.VectorSubcoreMesh(core_axis_name="c", subcore_axis_name="s",
                               num_cores=2, num_subcores=16)


@functools.partial(
    pl.kernel,
    out_type=jax.ShapeDtypeStruct((_N, _DIM), jnp.float32),
    mesh=_mesh,
    scratch_types=[
        pltpu.VMEM((_NCH, _CHUNK), jnp.int32),
    ] + [pltpu.VMEM((_CHUNK, _DIM), jnp.float32) for _ in range(_NCH)] + [
        pltpu.SemaphoreType.DMA,
        pltpu.SemaphoreType.DMA,
    ],
    compiler_params=pltpu.CompilerParams(use_tc_tiling_on_sc=False),
)
def _gather_kernel(cl_hbm, table_hbm, out_hbm, idx_v, *rest):
    bufs, (sem_g, sem_w) = rest[:_NCH], rest[_NCH:]
    wid = lax.axis_index("s") * 2 + lax.axis_index("c")
    base = wid * _BPW
    pltpu.sync_copy(cl_hbm.at[pl.ds(wid * _NCH, _NCH)], idx_v)
    gcps = [pltpu.async_copy(table_hbm.at[idx_v.at[c]], bufs[c], sem_g)
            for c in range(_NCH)]
    wcps = []
    for c in range(_NCH):
        gcps[c].wait()
        wcps.append(pltpu.async_copy(
            bufs[c], out_hbm.at[pl.ds(base + c * _CHUNK, _CHUNK)], sem_w))
    for w in wcps:
        w.wait()


def kernel(z, embeddings, W, b):
    b2 = b.reshape(1, _PDIM)
    emb_p, embn = pl.pallas_call(
        _embproj_kernel,
        out_shape=[
            jax.ShapeDtypeStruct((_NUM_EMB, _PDIM), jnp.float32),
            jax.ShapeDtypeStruct((1, _NUM_EMB), jnp.float32),
        ],
    )(embeddings, W, b2)
    one_hot, cl = pl.pallas_call(
        _closest_kernel,
        grid=(_N // _R,),
        in_specs=[
            pl.BlockSpec((_R, _DIM), lambda i: (i, 0)),
            pl.BlockSpec((_NUM_EMB, _PDIM), lambda i: (0, 0)),
            pl.BlockSpec((1, _NUM_EMB), lambda i: (0, 0)),
            pl.BlockSpec((_PDIM, _DIM), lambda i: (0, 0)),
            pl.BlockSpec((1, _PDIM), lambda i: (0, 0)),
        ],
        out_specs=[
            pl.BlockSpec((_R, _NUM_EMB), lambda i: (i, 0)),
            pl.BlockSpec((_R // _CHUNK, _CHUNK), lambda i: (i, 0)),
        ],
        out_shape=[
            jax.ShapeDtypeStruct((_N, _NUM_EMB), jnp.float32),
            jax.ShapeDtypeStruct((_N // _CHUNK, _CHUNK), jnp.int32),
        ],
    )(z, emb_p, embn, W, b2)
    quantized = _gather_kernel(cl, embeddings)
    return (quantized, one_hot)


# speedup vs baseline: 1.0792x; 1.0148x over previous
import functools

import jax
import jax.numpy as jnp
from jax import lax
from jax.experimental import pallas as pl
from jax.experimental.pallas import tpu as pltpu
from jax.experimental.pallas import tpu_sc as plsc

_NUM_EMB = 1024
_DIM = 64
_PDIM = 32
_N = 32768
_R = 2048

_NW = 32
_BPW = _N // _NW
_CHUNK = 128
_NCH = _BPW // _CHUNK


def _embproj_kernel(emb_ref, w_ref, b_ref, embp_ref, embn_ref):
    emb_p = jax.lax.dot_general(
        emb_ref[:], w_ref[:], (((1,), (1,)), ((), ())),
        preferred_element_type=jnp.float32) + b_ref[:]
    embp_ref[:] = emb_p
    embn_ref[:] = jnp.sum(emb_p * emb_p, axis=1)[None, :]


def _closest_kernel(z_ref, embp_ref, embn_ref, w_ref, b_ref, oh_ref, cl_ref):
    z_p = jax.lax.dot_general(
        z_ref[:], w_ref[:], (((1,), (1,)), ((), ())),
        preferred_element_type=jnp.float32) + b_ref[:]
    rowsq = jnp.sum(z_p * z_p, axis=1, keepdims=True)
    cross = jax.lax.dot_general(
        z_p, embp_ref[:], (((1,), (1,)), ((), ())),
        preferred_element_type=jnp.float32)
    dist = (rowsq + embn_ref[:]) - 2.0 * cross
    minv = jnp.min(dist, axis=1, keepdims=True)
    iota = jax.lax.broadcasted_iota(jnp.int32, (_R, _NUM_EMB), 1)
    masked = jnp.where(dist == minv, iota, jnp.int32(2 ** 30))
    closest = jnp.min(masked, axis=1)
    oh_ref[:] = (iota == closest[:, None]).astype(jnp.float32)
    cl_ref[:, :] = closest.reshape(_R // _CHUNK, _CHUNK)


_mesh = plsc.VectorSubcoreMesh(core_axis_name="c", subcore_axis_name="s",
                               num_cores=2, num_subcores=16)


@functools.partial(
    pl.kernel,
    out_type=jax.ShapeDtypeStruct((_N, _DIM), jnp.float32),
    mesh=_mesh,
    scratch_types=[
        pltpu.VMEM((_NCH, _CHUNK), jnp.int32),
    ] + [pltpu.VMEM((_CHUNK, _DIM), jnp.float32) for _ in range(_NCH)] + [
        pltpu.SemaphoreType.DMA,
        pltpu.SemaphoreType.DMA,
    ],
    compiler_params=pltpu.CompilerParams(use_tc_tiling_on_sc=False),
)
def _gather_kernel(cl_hbm, table_hbm, out_hbm, idx_v, *rest):
    bufs, (sem_g, sem_w) = rest[:_NCH], rest[_NCH:]
    wid = lax.axis_index("s") * 2 + lax.axis_index("c")
    base = wid * _BPW
    pltpu.sync_copy(cl_hbm.at[pl.ds(wid * _NCH, _NCH)], idx_v)
    gcps = [pltpu.async_copy(table_hbm.at[idx_v.at[c]], bufs[c], sem_g)
            for c in range(_NCH)]
    wcps = []
    for c in range(_NCH):
        gcps[c].wait()
        wcps.append(pltpu.async_copy(
            bufs[c], out_hbm.at[pl.ds(base + c * _CHUNK, _CHUNK)], sem_w))
    for w in wcps:
        w.wait()


def kernel(z, embeddings, W, b):
    b2 = b.reshape(1, _PDIM)
    emb_p, embn = pl.pallas_call(
        _embproj_kernel,
        out_shape=[
            jax.ShapeDtypeStruct((_NUM_EMB, _PDIM), jnp.float32),
            jax.ShapeDtypeStruct((1, _NUM_EMB), jnp.float32),
        ],
    )(embeddings, W, b2)
    one_hot, cl = pl.pallas_call(
        _closest_kernel,
        grid=(_N // _R,),
        in_specs=[
            pl.BlockSpec((_R, _DIM), lambda i: (i, 0)),
            pl.BlockSpec((_NUM_EMB, _PDIM), lambda i: (0, 0)),
            pl.BlockSpec((1, _NUM_EMB), lambda i: (0, 0)),
            pl.BlockSpec((_PDIM, _DIM), lambda i: (0, 0)),
            pl.BlockSpec((1, _PDIM), lambda i: (0, 0)),
        ],
        out_specs=[
            pl.BlockSpec((_R, _NUM_EMB), lambda i: (i, 0)),
            pl.BlockSpec((_R // _CHUNK, _CHUNK), lambda i: (i, 0)),
        ],
        out_shape=[
            jax.ShapeDtypeStruct((_N, _NUM_EMB), jnp.float32),
            jax.ShapeDtypeStruct((_N // _CHUNK, _CHUNK), jnp.int32),
        ],
    )(z, emb_p, embn, W, b2)
    quantized = _gather_kernel(cl, embeddings)
    return (quantized, one_hot)
